# baseline (speedup 1.0000x reference)
import jax
import jax.numpy as jnp
from jax import lax
from jax.experimental import pallas as pl
from jax.experimental.pallas import tpu as pltpu

N = 4096
IN = 512
OUT = 256
HEADS = 4
HD = OUT // HEADS

_BF = jnp.bfloat16
_F32 = jnp.float32

_BM1 = 1024
_BM2 = 256
_BM3 = 256

_TRANS_RHS = (((1,), (1,)), ((), ()))


def _dot(a, b):
    return jnp.dot(a, b, preferred_element_type=_F32)


def _dot_t(a, b):
    return lax.dot_general(a, b, _TRANS_RHS, preferred_element_type=_F32)


def _proj_body(feat_ref, feata_ref, ws_ref, wf_ref, sw_ref, fw_ref,
               s_ref, f_ref):
    sw = sw_ref[0]
    fw = fw_ref[0]
    x = feat_ref[...]
    xa = feata_ref[...]
    ws = ws_ref[...]
    wf = wf_ref[...]
    s_ref[...] = (jnp.concatenate([_dot(x, ws), _dot(xa, ws)], axis=1)
                  * sw).astype(_BF)
    f_ref[...] = (jnp.concatenate([_dot(x, wf), _dot(xa, wf)], axis=1)
                  * fw).astype(_BF)


def _agg_body(adj_ref, adjf_ref, s_ref, f_ref,
              wo_ref, uo_ref, fcw_ref, fcb_ref, wback_ref,
              emb_ref, emba_ref, h_ref, e_ref):
    zs_full = _dot(adj_ref[...].astype(_BF), s_ref[...])
    zf_full = _dot(adjf_ref[...].astype(_BF), f_ref[...])

    fcb = fcb_ref[...]
    wflat = wo_ref[...]
    rows = lax.broadcasted_iota(jnp.int32, (OUT, 128), 0)
    cols = lax.broadcasted_iota(jnp.int32, (OUT, 128), 1)
    ublk = jnp.where(cols == rows // HD, uo_ref[...], 0.0)

    def scores(zb):
        return _dot(jnp.tanh(_dot(zb, wflat)), ublk)

    def mha(zsb, zfb):
        s0 = scores(zsb)
        s1 = scores(zfb)
        m = jnp.maximum(s0, s1)
        e0 = jnp.exp(s0 - m)
        e1 = jnp.exp(s1 - m)
        den = e0 + e1
        a0 = e0 / den
        a1 = e1 / den
        parts = []
        for h in range(HEADS):
            parts.append(a0[:, h:h + 1] * zsb + a1[:, h:h + 1] * zfb)
        cat = jnp.concatenate(parts, axis=1)
        return _dot_t(cat, fcw_ref[...]) + fcb

    emb = mha(zs_full[:, :OUT], zf_full[:, :OUT])
    emba = mha(zs_full[:, OUT:], zf_full[:, OUT:])
    emb_ref[...] = emb
    emba_ref[...] = emba
    h_ref[...] = _dot(emb, wback_ref[...])
    e_ref[...] = jnp.concatenate([emb, emba], axis=1).astype(_BF)


def _read_body(gn0_ref, gn1_ref, gn2_ref, gn3_ref, e_ref, emb_ref,
               emba_ref, dw_ref, db_ref, ret_ref, reta_ref):
    e_all = e_ref[...]
    q = N // 4
    gns = [gn0_ref[...], gn1_ref[...], gn2_ref[...], gn3_ref[...]]
    vs = sum(_dot(g.astype(_BF), e_all[i * q:(i + 1) * q, :])
             for i, g in enumerate(gns))
    rs = sum(jnp.sum(g, axis=1, keepdims=True) for g in gns)
    db = db_ref[0]

    emb = emb_ref[...]
    emba = emba_ref[...]
    dw = dw_ref[...]

    def readg(v):
        g = v / rs
        nrm = jnp.maximum(
            jnp.sqrt(jnp.sum(g * g, axis=1, keepdims=True)), 1e-12)
        return jax.nn.sigmoid(g / nrm)

    g1 = readg(vs[:, :OUT])
    g2 = readg(vs[:, OUT:])
    cg = _dot_t(g1, dw)
    cga = _dot_t(g2, dw)
    sc1 = jnp.sum(emb * cg, axis=1, keepdims=True) + db
    sc2 = jnp.sum(emba * cg, axis=1, keepdims=True) + db
    sa1 = jnp.sum(emba * cga, axis=1, keepdims=True) + db
    sa2 = jnp.sum(emb * cga, axis=1, keepdims=True) + db
    ret_ref[...] = jnp.transpose(jnp.concatenate([sc1, sc2], axis=1))
    reta_ref[...] = jnp.transpose(jnp.concatenate([sa1, sa2], axis=1))


def kernel(feat, feat_a, adj, adj_feat, graph_neigh, weight_spatial,
           weight_feat, weight_back, spatial_weight, feature_weight,
           w_omega, u_omega, fc_w, fc_b, disc_w, disc_b):
    wflat = jnp.transpose(w_omega, (1, 0, 2)).reshape(OUT, OUT)
    uflat = u_omega.reshape(HEADS * HD, 1)
    fcb2 = fc_b.reshape(1, OUT)
    sw = spatial_weight.reshape(1)
    fw = feature_weight.reshape(1)
    db = disc_b.reshape(1)
    dw = disc_w[0]

    s, f = pl.pallas_call(
        _proj_body,
        grid=(N // _BM1,),
        in_specs=[
            pl.BlockSpec((_BM1, IN), lambda m: (m, 0)),
            pl.BlockSpec((_BM1, IN), lambda m: (m, 0)),
            pl.BlockSpec((IN, OUT), lambda m: (0, 0)),
            pl.BlockSpec((IN, OUT), lambda m: (0, 0)),
            pl.BlockSpec(memory_space=pltpu.SMEM),
            pl.BlockSpec(memory_space=pltpu.SMEM),
        ],
        out_specs=[
            pl.BlockSpec((_BM1, 2 * OUT), lambda m: (m, 0)),
            pl.BlockSpec((_BM1, 2 * OUT), lambda m: (m, 0)),
        ],
        out_shape=[
            jax.ShapeDtypeStruct((N, 2 * OUT), _BF),
            jax.ShapeDtypeStruct((N, 2 * OUT), _BF),
        ],
        compiler_params=pltpu.CompilerParams(
            dimension_semantics=("parallel",)),
    )(feat, feat_a, weight_spatial, weight_feat, sw, fw)

    emb, emba, h, e = pl.pallas_call(
        _agg_body,
        grid=(N // _BM2,),
        in_specs=[
            pl.BlockSpec((_BM2, N), lambda m: (m, 0)),
            pl.BlockSpec((_BM2, N), lambda m: (m, 0)),
            pl.BlockSpec((N, 2 * OUT), lambda m: (0, 0)),
            pl.BlockSpec((N, 2 * OUT), lambda m: (0, 0)),
            pl.BlockSpec((OUT, OUT), lambda m: (0, 0)),
            pl.BlockSpec((OUT, 1), lambda m: (0, 0)),
            pl.BlockSpec((OUT, 4 * OUT), lambda m: (0, 0)),
            pl.BlockSpec((1, OUT), lambda m: (0, 0)),
            pl.BlockSpec((OUT, IN), lambda m: (0, 0)),
        ],
        out_specs=[
            pl.BlockSpec((_BM2, OUT), lambda m: (m, 0)),
            pl.BlockSpec((_BM2, OUT), lambda m: (m, 0)),
            pl.BlockSpec((_BM2, IN), lambda m: (m, 0)),
            pl.BlockSpec((_BM2, 2 * OUT), lambda m: (m, 0)),
        ],
        out_shape=[
            jax.ShapeDtypeStruct((N, OUT), _F32),
            jax.ShapeDtypeStruct((N, OUT), _F32),
            jax.ShapeDtypeStruct((N, IN), _F32),
            jax.ShapeDtypeStruct((N, 2 * OUT), _BF),
        ],
        compiler_params=pltpu.CompilerParams(
            dimension_semantics=("parallel",),
            vmem_limit_bytes=100 * 1024 * 1024),
    )(adj, adj_feat, s, f, wflat, uflat, fc_w, fcb2, weight_back)

    ret, reta = pl.pallas_call(
        _read_body,
        grid=(N // _BM3,),
        in_specs=[
            pl.BlockSpec((_BM3, N // 4), lambda m: (m, 0)),
            pl.BlockSpec((_BM3, N // 4), lambda m: (m, 1)),
            pl.BlockSpec((_BM3, N // 4), lambda m: (m, 2)),
            pl.BlockSpec((_BM3, N // 4), lambda m: (m, 3)),
            pl.BlockSpec((N, 2 * OUT), lambda m: (0, 0)),
            pl.BlockSpec((_BM3, OUT), lambda m: (m, 0)),
            pl.BlockSpec((_BM3, OUT), lambda m: (m, 0)),
            pl.BlockSpec((OUT, OUT), lambda m: (0, 0)),
            pl.BlockSpec(memory_space=pltpu.SMEM),
        ],
        out_specs=[
            pl.BlockSpec((2, _BM3), lambda m: (0, m)),
            pl.BlockSpec((2, _BM3), lambda m: (0, m)),
        ],
        out_shape=[
            jax.ShapeDtypeStruct((2, N), _F32),
            jax.ShapeDtypeStruct((2, N), _F32),
        ],
        compiler_params=pltpu.CompilerParams(
            dimension_semantics=("parallel",),
            vmem_limit_bytes=100 * 1024 * 1024),
    )(graph_neigh, graph_neigh, graph_neigh, graph_neigh, e, emb, emba, dw, db)

    return (emb, h, ret.T, reta.T)


# speedup vs baseline: 1.0664x; 1.0664x over previous
import jax
import jax.numpy as jnp
from jax import lax
from jax.experimental import pallas as pl
from jax.experimental.pallas import tpu as pltpu

N = 4096
IN = 512
OUT = 256
HEADS = 4
HD = OUT // HEADS

_BF = jnp.bfloat16
_F32 = jnp.float32

_BM1 = 1024
_BM2 = 512
_BM3 = 512

_TRANS_RHS = (((1,), (1,)), ((), ()))


def _dot(a, b):
    return jnp.dot(a, b, preferred_element_type=_F32)


def _dot_t(a, b):
    return lax.dot_general(a, b, _TRANS_RHS, preferred_element_type=_F32)


def _proj_body(feat_ref, feata_ref, ws_ref, wf_ref, sw_ref, fw_ref,
               s_ref, f_ref):
    sw = sw_ref[0]
    fw = fw_ref[0]
    x = feat_ref[...]
    xa = feata_ref[...]
    ws = ws_ref[...]
    wf = wf_ref[...]
    s_ref[...] = (jnp.concatenate([_dot(x, ws), _dot(xa, ws)], axis=1)
                  * sw).astype(_BF)
    f_ref[...] = (jnp.concatenate([_dot(x, wf), _dot(xa, wf)], axis=1)
                  * fw).astype(_BF)


def _agg_body(adj_ref, adjf_ref, s_ref, f_ref,
              wo_ref, uo_ref, fcw_ref, fcb_ref, wback_ref,
              emb_ref, emba_ref, h_ref, e_ref):
    zs_full = _dot(adj_ref[...].astype(_BF), s_ref[...])
    zf_full = _dot(adjf_ref[...].astype(_BF), f_ref[...])

    fcb = fcb_ref[...]
    wflat = wo_ref[...]
    rows = lax.broadcasted_iota(jnp.int32, (OUT, 128), 0)
    cols = lax.broadcasted_iota(jnp.int32, (OUT, 128), 1)
    ublk = jnp.where(cols == rows // HD, uo_ref[...], 0.0)

    def scores(zb):
        return _dot(jnp.tanh(_dot(zb, wflat)), ublk)

    def mha(zsb, zfb):
        s0 = scores(zsb)
        s1 = scores(zfb)
        m = jnp.maximum(s0, s1)
        e0 = jnp.exp(s0 - m)
        e1 = jnp.exp(s1 - m)
        den = e0 + e1
        a0 = e0 / den
        a1 = e1 / den
        parts = []
        for h in range(HEADS):
            parts.append(a0[:, h:h + 1] * zsb + a1[:, h:h + 1] * zfb)
        cat = jnp.concatenate(parts, axis=1)
        return _dot_t(cat, fcw_ref[...]) + fcb

    emb = mha(zs_full[:, :OUT], zf_full[:, :OUT])
    emba = mha(zs_full[:, OUT:], zf_full[:, OUT:])
    emb_ref[...] = emb
    emba_ref[...] = emba
    h_ref[...] = _dot(emb, wback_ref[...])
    e_ref[...] = jnp.concatenate([emb, emba], axis=1).astype(_BF)


def _read_body(gnl_ref, gnr_ref, e_ref, emb_ref, emba_ref, dw_ref, db_ref,
               ret_ref, reta_ref):
    gnl = gnl_ref[...]
    gnr = gnr_ref[...]
    e_all = e_ref[...]
    vs = (_dot(gnl.astype(_BF), e_all[: N // 2, :]) +
          _dot(gnr.astype(_BF), e_all[N // 2:, :]))
    rs = (jnp.sum(gnl, axis=1, keepdims=True) +
          jnp.sum(gnr, axis=1, keepdims=True))
    db = db_ref[0]

    emb = emb_ref[...]
    emba = emba_ref[...]
    dw = dw_ref[...]

    def readg(v):
        g = v / rs
        nrm = jnp.maximum(
            jnp.sqrt(jnp.sum(g * g, axis=1, keepdims=True)), 1e-12)
        return jax.nn.sigmoid(g / nrm)

    g1 = readg(vs[:, :OUT])
    g2 = readg(vs[:, OUT:])
    cg = _dot_t(g1, dw)
    cga = _dot_t(g2, dw)
    sc1 = jnp.sum(emb * cg, axis=1, keepdims=True) + db
    sc2 = jnp.sum(emba * cg, axis=1, keepdims=True) + db
    sa1 = jnp.sum(emba * cga, axis=1, keepdims=True) + db
    sa2 = jnp.sum(emb * cga, axis=1, keepdims=True) + db
    ret_ref[...] = jnp.transpose(jnp.concatenate([sc1, sc2], axis=1))
    reta_ref[...] = jnp.transpose(jnp.concatenate([sa1, sa2], axis=1))


def kernel(feat, feat_a, adj, adj_feat, graph_neigh, weight_spatial,
           weight_feat, weight_back, spatial_weight, feature_weight,
           w_omega, u_omega, fc_w, fc_b, disc_w, disc_b):
    wflat = jnp.transpose(w_omega, (1, 0, 2)).reshape(OUT, OUT)
    uflat = u_omega.reshape(HEADS * HD, 1)
    fcb2 = fc_b.reshape(1, OUT)
    sw = spatial_weight.reshape(1)
    fw = feature_weight.reshape(1)
    db = disc_b.reshape(1)
    dw = disc_w[0]

    s, f = pl.pallas_call(
        _proj_body,
        grid=(N // _BM1,),
        in_specs=[
            pl.BlockSpec((_BM1, IN), lambda m: (m, 0)),
            pl.BlockSpec((_BM1, IN), lambda m: (m, 0)),
            pl.BlockSpec((IN, OUT), lambda m: (0, 0)),
            pl.BlockSpec((IN, OUT), lambda m: (0, 0)),
            pl.BlockSpec(memory_space=pltpu.SMEM),
            pl.BlockSpec(memory_space=pltpu.SMEM),
        ],
        out_specs=[
            pl.BlockSpec((_BM1, 2 * OUT), lambda m: (m, 0)),
            pl.BlockSpec((_BM1, 2 * OUT), lambda m: (m, 0)),
        ],
        out_shape=[
            jax.ShapeDtypeStruct((N, 2 * OUT), _BF),
            jax.ShapeDtypeStruct((N, 2 * OUT), _BF),
        ],
        compiler_params=pltpu.CompilerParams(
            dimension_semantics=("parallel",)),
    )(feat, feat_a, weight_spatial, weight_feat, sw, fw)

    emb, emba, h, e = pl.pallas_call(
        _agg_body,
        grid=(N // _BM2,),
        in_specs=[
            pl.BlockSpec((_BM2, N), lambda m: (m, 0)),
            pl.BlockSpec((_BM2, N), lambda m: (m, 0)),
            pl.BlockSpec((N, 2 * OUT), lambda m: (0, 0)),
            pl.BlockSpec((N, 2 * OUT), lambda m: (0, 0)),
            pl.BlockSpec((OUT, OUT), lambda m: (0, 0)),
            pl.BlockSpec((OUT, 1), lambda m: (0, 0)),
            pl.BlockSpec((OUT, 4 * OUT), lambda m: (0, 0)),
            pl.BlockSpec((1, OUT), lambda m: (0, 0)),
            pl.BlockSpec((OUT, IN), lambda m: (0, 0)),
        ],
        out_specs=[
            pl.BlockSpec((_BM2, OUT), lambda m: (m, 0)),
            pl.BlockSpec((_BM2, OUT), lambda m: (m, 0)),
            pl.BlockSpec((_BM2, IN), lambda m: (m, 0)),
            pl.BlockSpec((_BM2, 2 * OUT), lambda m: (m, 0)),
        ],
        out_shape=[
            jax.ShapeDtypeStruct((N, OUT), _F32),
            jax.ShapeDtypeStruct((N, OUT), _F32),
            jax.ShapeDtypeStruct((N, IN), _F32),
            jax.ShapeDtypeStruct((N, 2 * OUT), _BF),
        ],
        compiler_params=pltpu.CompilerParams(
            dimension_semantics=("parallel",),
            vmem_limit_bytes=100 * 1024 * 1024),
    )(adj, adj_feat, s, f, wflat, uflat, fc_w, fcb2, weight_back)

    ret, reta = pl.pallas_call(
        _read_body,
        grid=(N // _BM3,),
        in_specs=[
            pl.BlockSpec((_BM3, N // 2), lambda m: (m, 0)),
            pl.BlockSpec((_BM3, N // 2), lambda m: (m, 1)),
            pl.BlockSpec((N, 2 * OUT), lambda m: (0, 0)),
            pl.BlockSpec((_BM3, OUT), lambda m: (m, 0)),
            pl.BlockSpec((_BM3, OUT), lambda m: (m, 0)),
            pl.BlockSpec((OUT, OUT), lambda m: (0, 0)),
            pl.BlockSpec(memory_space=pltpu.SMEM),
        ],
        out_specs=[
            pl.BlockSpec((2, _BM3), lambda m: (0, m)),
            pl.BlockSpec((2, _BM3), lambda m: (0, m)),
        ],
        out_shape=[
            jax.ShapeDtypeStruct((2, N), _F32),
            jax.ShapeDtypeStruct((2, N), _F32),
        ],
        compiler_params=pltpu.CompilerParams(
            dimension_semantics=("parallel",),
            vmem_limit_bytes=100 * 1024 * 1024),
    )(graph_neigh, graph_neigh, e, emb, emba, dw, db)

    return (emb, h, ret.T, reta.T)


# speedup vs baseline: 1.0764x; 1.0094x over previous
import jax
import jax.numpy as jnp
from jax import lax
from jax.experimental import pallas as pl
from jax.experimental.pallas import tpu as pltpu

N = 4096
IN = 512
OUT = 256
HEADS = 4
HD = OUT // HEADS

_BF = jnp.bfloat16
_F32 = jnp.float32

_BM1 = 2048
_BM2 = 512
_BM3 = 512

_TRANS_RHS = (((1,), (1,)), ((), ()))


def _dot(a, b):
    return jnp.dot(a, b, preferred_element_type=_F32)


def _dot_t(a, b):
    return lax.dot_general(a, b, _TRANS_RHS, preferred_element_type=_F32)


def _proj_body(feat_ref, feata_ref, ws_ref, wf_ref, sw_ref, fw_ref,
               s_ref, f_ref):
    sw = sw_ref[0]
    fw = fw_ref[0]
    x = feat_ref[...]
    xa = feata_ref[...]
    ws = ws_ref[...]
    wf = wf_ref[...]
    s_ref[...] = (jnp.concatenate([_dot(x, ws), _dot(xa, ws)], axis=1)
                  * sw).astype(_BF)
    f_ref[...] = (jnp.concatenate([_dot(x, wf), _dot(xa, wf)], axis=1)
                  * fw).astype(_BF)


def _agg_body(adj_ref, adjf_ref, s_ref, f_ref,
              wo_ref, uo_ref, fcw_ref, fcb_ref, wback_ref,
              emb_ref, emba_ref, h_ref, e_ref):
    zs_full = _dot(adj_ref[...].astype(_BF), s_ref[...])
    zf_full = _dot(adjf_ref[...].astype(_BF), f_ref[...])

    fcb = fcb_ref[...]
    wflat = wo_ref[...]
    rows = lax.broadcasted_iota(jnp.int32, (OUT, 128), 0)
    cols = lax.broadcasted_iota(jnp.int32, (OUT, 128), 1)
    ublk = jnp.where(cols == rows // HD, uo_ref[...], 0.0)

    def scores(zb):
        return _dot(jnp.tanh(_dot(zb, wflat)), ublk)

    def mha(zsb, zfb):
        s0 = scores(zsb)
        s1 = scores(zfb)
        m = jnp.maximum(s0, s1)
        e0 = jnp.exp(s0 - m)
        e1 = jnp.exp(s1 - m)
        den = e0 + e1
        a0 = e0 / den
        a1 = e1 / den
        parts = []
        for h in range(HEADS):
            parts.append(a0[:, h:h + 1] * zsb + a1[:, h:h + 1] * zfb)
        cat = jnp.concatenate(parts, axis=1)
        return _dot_t(cat, fcw_ref[...]) + fcb

    emb = mha(zs_full[:, :OUT], zf_full[:, :OUT])
    emba = mha(zs_full[:, OUT:], zf_full[:, OUT:])
    emb_ref[...] = emb
    emba_ref[...] = emba
    h_ref[...] = _dot(emb, wback_ref[...])
    e_ref[...] = jnp.concatenate([emb, emba], axis=1).astype(_BF)


def _read_body(gnl_ref, gnr_ref, e_ref, emb_ref, emba_ref, dw_ref, db_ref,
               ret_ref, reta_ref):
    gnl = gnl_ref[...]
    gnr = gnr_ref[...]
    e_all = e_ref[...]
    vs = (_dot(gnl.astype(_BF), e_all[: N // 2, :]) +
          _dot(gnr.astype(_BF), e_all[N // 2:, :]))
    rs = (jnp.sum(gnl, axis=1, keepdims=True) +
          jnp.sum(gnr, axis=1, keepdims=True))
    db = db_ref[0]

    emb = emb_ref[...]
    emba = emba_ref[...]
    dw = dw_ref[...]

    def readg(v):
        g = v / rs
        nrm = jnp.maximum(
            jnp.sqrt(jnp.sum(g * g, axis=1, keepdims=True)), 1e-12)
        return jax.nn.sigmoid(g / nrm)

    g1 = readg(vs[:, :OUT])
    g2 = readg(vs[:, OUT:])
    cg = _dot_t(g1, dw)
    cga = _dot_t(g2, dw)
    sc1 = jnp.sum(emb * cg, axis=1, keepdims=True) + db
    sc2 = jnp.sum(emba * cg, axis=1, keepdims=True) + db
    sa1 = jnp.sum(emba * cga, axis=1, keepdims=True) + db
    sa2 = jnp.sum(emb * cga, axis=1, keepdims=True) + db
    ret_ref[...] = jnp.transpose(jnp.concatenate([sc1, sc2], axis=1))
    reta_ref[...] = jnp.transpose(jnp.concatenate([sa1, sa2], axis=1))


def kernel(feat, feat_a, adj, adj_feat, graph_neigh, weight_spatial,
           weight_feat, weight_back, spatial_weight, feature_weight,
           w_omega, u_omega, fc_w, fc_b, disc_w, disc_b):
    wflat = jnp.transpose(w_omega, (1, 0, 2)).reshape(OUT, OUT)
    uflat = u_omega.reshape(HEADS * HD, 1)
    fcb2 = fc_b.reshape(1, OUT)
    sw = spatial_weight.reshape(1)
    fw = feature_weight.reshape(1)
    db = disc_b.reshape(1)
    dw = disc_w[0]

    s, f = pl.pallas_call(
        _proj_body,
        grid=(N // _BM1,),
        in_specs=[
            pl.BlockSpec((_BM1, IN), lambda m: (m, 0)),
            pl.BlockSpec((_BM1, IN), lambda m: (m, 0)),
            pl.BlockSpec((IN, OUT), lambda m: (0, 0)),
            pl.BlockSpec((IN, OUT), lambda m: (0, 0)),
            pl.BlockSpec(memory_space=pltpu.SMEM),
            pl.BlockSpec(memory_space=pltpu.SMEM),
        ],
        out_specs=[
            pl.BlockSpec((_BM1, 2 * OUT), lambda m: (m, 0)),
            pl.BlockSpec((_BM1, 2 * OUT), lambda m: (m, 0)),
        ],
        out_shape=[
            jax.ShapeDtypeStruct((N, 2 * OUT), _BF),
            jax.ShapeDtypeStruct((N, 2 * OUT), _BF),
        ],
        compiler_params=pltpu.CompilerParams(
            dimension_semantics=("parallel",)),
    )(feat, feat_a, weight_spatial, weight_feat, sw, fw)

    emb, emba, h, e = pl.pallas_call(
        _agg_body,
        grid=(N // _BM2,),
        in_specs=[
            pl.BlockSpec((_BM2, N), lambda m: (m, 0)),
            pl.BlockSpec((_BM2, N), lambda m: (m, 0)),
            pl.BlockSpec((N, 2 * OUT), lambda m: (0, 0)),
            pl.BlockSpec((N, 2 * OUT), lambda m: (0, 0)),
            pl.BlockSpec((OUT, OUT), lambda m: (0, 0)),
            pl.BlockSpec((OUT, 1), lambda m: (0, 0)),
            pl.BlockSpec((OUT, 4 * OUT), lambda m: (0, 0)),
            pl.BlockSpec((1, OUT), lambda m: (0, 0)),
            pl.BlockSpec((OUT, IN), lambda m: (0, 0)),
        ],
        out_specs=[
            pl.BlockSpec((_BM2, OUT), lambda m: (m, 0)),
            pl.BlockSpec((_BM2, OUT), lambda m: (m, 0)),
            pl.BlockSpec((_BM2, IN), lambda m: (m, 0)),
            pl.BlockSpec((_BM2, 2 * OUT), lambda m: (m, 0)),
        ],
        out_shape=[
            jax.ShapeDtypeStruct((N, OUT), _F32),
            jax.ShapeDtypeStruct((N, OUT), _F32),
            jax.ShapeDtypeStruct((N, IN), _F32),
            jax.ShapeDtypeStruct((N, 2 * OUT), _BF),
        ],
        compiler_params=pltpu.CompilerParams(
            dimension_semantics=("parallel",),
            vmem_limit_bytes=100 * 1024 * 1024),
    )(adj, adj_feat, s, f, wflat, uflat, fc_w, fcb2, weight_back)

    ret, reta = pl.pallas_call(
        _read_body,
        grid=(N // _BM3,),
        in_specs=[
            pl.BlockSpec((_BM3, N // 2), lambda m: (m, 0)),
            pl.BlockSpec((_BM3, N // 2), lambda m: (m, 1)),
            pl.BlockSpec((N, 2 * OUT), lambda m: (0, 0)),
            pl.BlockSpec((_BM3, OUT), lambda m: (m, 0)),
            pl.BlockSpec((_BM3, OUT), lambda m: (m, 0)),
            pl.BlockSpec((OUT, OUT), lambda m: (0, 0)),
            pl.BlockSpec(memory_space=pltpu.SMEM),
        ],
        out_specs=[
            pl.BlockSpec((2, _BM3), lambda m: (0, m)),
            pl.BlockSpec((2, _BM3), lambda m: (0, m)),
        ],
        out_shape=[
            jax.ShapeDtypeStruct((2, N), _F32),
            jax.ShapeDtypeStruct((2, N), _F32),
        ],
        compiler_params=pltpu.CompilerParams(
            dimension_semantics=("parallel",),
            vmem_limit_bytes=100 * 1024 * 1024),
    )(graph_neigh, graph_neigh, e, emb, emba, dw, db)

    return (emb, h, ret.T, reta.T)
